# baseline (speedup 1.0000x reference)
import jax
import jax.numpy as jnp
from jax import lax
from jax.experimental import pallas as pl
from jax.experimental.pallas import tpu as pltpu
from jax.experimental.pallas import tpu_sc as plsc

B = 1024
NB_IPRS = 26406
N = 10000
E = 160000
H = 512

KBLK = 512
KSTEPS = (NB_IPRS + KBLK - 1) // KBLK
KREM = NB_IPRS - (KSTEPS - 1) * KBLK

NBLK = 400
NSTEPS = N // NBLK

NCHUNK = 8
CW = H // NCHUNK

ER = E // 128
RPT = 80
RLAST = 50
CP = 624

N2 = 10240
NB2 = 512
NS2 = N2 // NB2


def _layer_norm(x, g, be):
    mu = jnp.mean(x, axis=-1, keepdims=True)
    var = jnp.mean(x * x, axis=-1, keepdims=True) - mu * mu
    return (x - mu) * lax.rsqrt(var + 1e-5) * g + be


def _mlp_kernel(f_ref, w1_ref, b1_ref, g1_ref, be1_ref, w2_ref, b2_ref,
                g2_ref, be2_ref, o_ref, acc_ref):
    k = pl.program_id(0)

    @pl.when(k == 0)
    def _():
        acc_ref[...] = jnp.zeros_like(acc_ref)

    lim = jnp.where(k == KSTEPS - 1, KREM, KBLK)
    col = lax.broadcasted_iota(jnp.int32, (B, KBLK), 1)
    row = lax.broadcasted_iota(jnp.int32, (KBLK, H), 0)
    fb = jnp.where(col < lim, f_ref[...], 0.0)
    wb = jnp.where(row < lim, w1_ref[...], 0.0)

    acc_ref[...] += jnp.dot(fb, wb, preferred_element_type=jnp.float32)

    @pl.when(k == KSTEPS - 1)
    def _():
        x1 = jax.nn.relu(acc_ref[...] + b1_ref[...])
        x1 = _layer_norm(x1, g1_ref[...], be1_ref[...])
        x2 = jax.nn.relu(
            jnp.dot(x1, w2_ref[...], preferred_element_type=jnp.float32)
            + b2_ref[...])
        x2 = _layer_norm(x2, g2_ref[...], be2_ref[...])
        o_ref[...] = x1 + x2


def _mlp_call(features, W1, b1, g1, be1, W2, b2, g2, be2):
    return pl.pallas_call(
        _mlp_kernel,
        grid=(KSTEPS,),
        in_specs=[
            pl.BlockSpec((B, KBLK), lambda k: (0, k)),
            pl.BlockSpec((KBLK, H), lambda k: (k, 0)),
        ] + [pl.BlockSpec((1, H), lambda k: (0, 0))] * 3
        + [pl.BlockSpec((H, H), lambda k: (0, 0))]
        + [pl.BlockSpec((1, H), lambda k: (0, 0))] * 3,
        out_specs=pl.BlockSpec((B, H), lambda k: (0, 0)),
        out_shape=jax.ShapeDtypeStruct((B, H), jnp.float32),
        scratch_shapes=[pltpu.VMEM((B, H), jnp.float32)],
    )(features, W1, b1, g1, be1, W2, b2, g2, be2)


def _gat_fc_kernel(ge_ref, w_ref, al_ref, ar_ref, *o_refs):
    m = pl.program_id(0)
    f_refs, el_ref, er_ref = o_refs[:NCHUNK], o_refs[NCHUNK], o_refs[NCHUNK + 1]
    fb = jnp.dot(ge_ref[...], w_ref[...], preferred_element_type=jnp.float32)
    for c in range(NCHUNK):
        f_refs[c][...] = fb[:, c * CW:(c + 1) * CW]
    el_ref[pl.ds(m, 1), :] = jnp.sum(fb * al_ref[...], axis=-1)[None, :]
    er_ref[pl.ds(m, 1), :] = jnp.sum(fb * ar_ref[...], axis=-1)[None, :]


def _gat_fc_call(go_embed, gat_W, attn_l, attn_r):
    chunk = jax.ShapeDtypeStruct((N, CW), jnp.float32)
    return pl.pallas_call(
        _gat_fc_kernel,
        grid=(NSTEPS,),
        in_specs=[
            pl.BlockSpec((NBLK, H), lambda m: (m, 0)),
            pl.BlockSpec((H, H), lambda m: (0, 0)),
            pl.BlockSpec((1, H), lambda m: (0, 0)),
            pl.BlockSpec((1, H), lambda m: (0, 0)),
        ],
        out_specs=[pl.BlockSpec((NBLK, CW), lambda m: (m, 0))] * NCHUNK
        + [pl.BlockSpec((NSTEPS, NBLK), lambda m: (0, 0))] * 2,
        out_shape=[chunk] * NCHUNK
        + [jax.ShapeDtypeStruct((NSTEPS, NBLK), jnp.float32)] * 2,
    )(go_embed, gat_W, attn_l, attn_r)


def _logits_kernel(x_ref, *refs):
    a_refs = refs[:NCHUNK]
    dn_ref, dn1_ref, gbias_ref, gob_ref, o_ref = refs[NCHUNK:]
    m = pl.program_id(0)
    gob = jnp.concatenate([a[...] for a in a_refs], axis=1)
    dn = dn_ref[...][:, 0:1] + dn1_ref[...][:, 0:1]
    gob = gob / (dn + 1e-9) + gbias_ref[...]
    z = lax.dot_general(x_ref[...], gob, (((1,), (1,)), ((), ())),
                        preferred_element_type=jnp.float32)
    o_ref[...] = jax.nn.sigmoid(z + gob_ref[pl.ds(m, 1), :])


def _logits_call(x, accs, denom16, denomb, gat_bias, go_bias):
    go_bias = jnp.pad(go_bias.reshape(N), (0, N2 - N)).reshape(NS2, NB2)
    out = pl.pallas_call(
        _logits_kernel,
        grid=(NS2,),
        in_specs=[pl.BlockSpec((B, H), lambda m: (0, 0))]
        + [pl.BlockSpec((NB2, CW), lambda m: (m, 0))] * NCHUNK
        + [
            pl.BlockSpec((NB2, CW), lambda m: (m, 0)),
            pl.BlockSpec((NB2, CW), lambda m: (m, 0)),
            pl.BlockSpec((1, H), lambda m: (0, 0)),
            pl.BlockSpec((NS2, NB2), lambda m: (0, 0)),
        ],
        out_specs=pl.BlockSpec((B, NB2), lambda m: (0, m)),
        out_shape=jax.ShapeDtypeStruct((B, N2), jnp.float32),
    )(x, *accs, denom16, denomb, gat_bias, go_bias)
    return out[:, :N]


def _splat(v):
    return jnp.full((16,), v, jnp.int32)


def _sc_body(ei_h, el_h, er_h, *refs):
    f_h = refs[:NCHUNK]
    a_h = refs[NCHUNK:2 * NCHUNK]
    den_h = refs[2 * NCHUNK]
    den1_h = refs[2 * NCHUNK + 1]
    (srcv, dstv, eev, elv, erv, buf, buf2, buf3, zbuf, acc_sh,
     sem, sem2, sem3, sem4, sem5, sem6) = refs[2 * NCHUNK + 2:]

    cid = lax.axis_index("c")
    sid = lax.axis_index("s")
    n = jnp.where(sid == 15, RLAST, RPT)
    p0 = sid * CP
    r0 = sid * RPT

    @pl.when(sid < 15)
    def _():
        pltpu.sync_copy(ei_h.at[0, pl.ds(r0, RPT)], srcv)
        pltpu.sync_copy(ei_h.at[1, pl.ds(r0, RPT)], dstv)

    @pl.when(sid == 15)
    def _():
        pltpu.sync_copy(ei_h.at[0, pl.ds(r0, RLAST)],
                        srcv.at[pl.ds(0, RLAST)])
        pltpu.sync_copy(ei_h.at[1, pl.ds(r0, RLAST)],
                        dstv.at[pl.ds(0, RLAST)])

    pltpu.sync_copy(el_h, elv)
    pltpu.sync_copy(er_h, erv)

    def ee_row(j, _):
        for i in range(8):
            sv = srcv[j, pl.ds(16 * i, 16)]
            dv = dstv[j, pl.ds(16 * i, 16)]
            e = (plsc.load_gather(elv, [sv >> 4, sv & 15])
                 + plsc.load_gather(erv, [dv >> 4, dv & 15]))
            e = jnp.maximum(e, 0.2 * e)
            eev[j, pl.ds(16 * i, 16)] = jnp.exp(e)
        return 0

    lax.fori_loop(0, n, ee_row, 0)

    def zb_row(i, _):
        for d in range(CW // 16):
            zbuf[i, pl.ds(16 * d, 16)] = jnp.zeros((16,), jnp.float32)
        return 0

    lax.fori_loop(0, 78, zb_row, 0)

    def zero_accsh():
        for t in range(8):
            pltpu.sync_copy(zbuf, acc_sh.at[pl.ds(p0 + 78 * t, 78)])

        @pl.when(sid == 15)
        def _():
            pltpu.sync_copy(zbuf.at[pl.ds(0, 16)],
                            acc_sh.at[pl.ds(N - 16, 16)])

    def scale_rows(b, j):
        def srow(g, _):
            r0 = g * 8
            for u in range(8):
                r = r0 + u
                eef = plsc.load_gather(eev, [_splat(j), _splat(r)])
                for d in range(CW // 16):
                    sl = pl.ds(16 * d, 16)
                    b[r, sl] = b[r, sl] * eef
            return 0

        lax.fori_loop(0, 16, srow, 0)

    def chunk_pass(feat_h):
        bufs = (buf, buf2, buf3)
        gsem = (sem, sem2, sem3)
        ssem = (sem4, sem5, sem6)

        def gissue(j, b):
            pltpu.async_copy(feat_h.at[srcv.at[j]], bufs[b], gsem[b])

        gissue(0, 0)
        gissue(1, 1)

        def grp(g, _):
            for u in range(3):
                j = 3 * g + u

                @pl.when(j < n)
                def _(j=j, u=u):
                    pltpu.make_async_copy(feat_h.at[srcv.at[j]], bufs[u],
                                          gsem[u]).wait()
                    scale_rows(bufs[u], j)
                    pltpu.async_copy(bufs[u], acc_sh.at[dstv.at[j]],
                                     ssem[u], add=True)

                @pl.when((j >= 1) & (j - 1 < n))
                def _(j=j, u=u):
                    b = (u + 2) % 3
                    pltpu.make_async_copy(bufs[b],
                                          acc_sh.at[dstv.at[j - 1]],
                                          ssem[b]).wait()

                @pl.when(j + 2 < n)
                def _(j=j, u=u):
                    gissue(j + 2, (u + 2) % 3)

            return 0

        lax.fori_loop(0, 27, grp, 0)

    def denom_pass(lo, hi):
        def blk(j, _):
            def row(g, _):
                for u in range(8):
                    r = g * 8 + u
                    eef = plsc.load_gather(eev, [_splat(j), _splat(r)])
                    for d in range(CW // 16):
                        buf[r, pl.ds(16 * d, 16)] = eef
                return 0

            lax.fori_loop(0, 16, row, 0)
            pltpu.sync_copy(buf, acc_sh.at[dstv.at[j]], add=True)
            return 0

        lax.fori_loop(lo, hi, blk, 0)

    def copy_out(src_sh, dst_ref):
        pltpu.sync_copy(src_sh.at[pl.ds(p0, CP)], dst_ref.at[pl.ds(p0, CP)])

        @pl.when(sid == 15)
        def _():
            pltpu.sync_copy(src_sh.at[pl.ds(N - 16, 16)],
                            dst_ref.at[pl.ds(N - 16, 16)])

    def run_sc(half, dref):
        for i in range(NCHUNK // 2):
            zero_accsh()
            plsc.subcore_barrier()
            chunk_pass(f_h[half * (NCHUNK // 2) + i])
            plsc.subcore_barrier()
            copy_out(acc_sh, a_h[half * (NCHUNK // 2) + i])
        zero_accsh()
        plsc.subcore_barrier()
        if half == 0:
            denom_pass(0, n // 2)
        else:
            denom_pass(n // 2, n)
        plsc.subcore_barrier()
        copy_out(acc_sh, dref)

    @pl.when(cid == 0)
    def _():
        run_sc(0, den_h)

    @pl.when(cid == 1)
    def _():
        run_sc(1, den1_h)


def _gat_aggregate(ei3, el, er, feats):
    mesh = plsc.VectorSubcoreMesh(core_axis_name="c", subcore_axis_name="s")
    chunk = jax.ShapeDtypeStruct((N2, CW), jnp.float32)
    out_type = [chunk] * NCHUNK + [jax.ShapeDtypeStruct((N2, CW), jnp.float32)] * 2
    fn = pl.kernel(
        _sc_body,
        out_type=out_type,
        mesh=mesh,
        compiler_params=pltpu.CompilerParams(needs_layout_passes=False,
                                             use_tc_tiling_on_sc=False),
        scratch_types=[
            pltpu.VMEM((RPT, 128), jnp.int32),
            pltpu.VMEM((RPT, 128), jnp.int32),
            pltpu.VMEM((RPT, 128), jnp.float32),
            pltpu.VMEM((N // 16, 16), jnp.float32),
            pltpu.VMEM((N // 16, 16), jnp.float32),
            pltpu.VMEM((128, CW), jnp.float32),
            pltpu.VMEM((128, CW), jnp.float32),
            pltpu.VMEM((128, CW), jnp.float32),
            pltpu.VMEM((78, CW), jnp.float32),
            pltpu.VMEM_SHARED((N, CW), jnp.float32),
            pltpu.SemaphoreType.DMA,
            pltpu.SemaphoreType.DMA,
            pltpu.SemaphoreType.DMA,
            pltpu.SemaphoreType.DMA,
            pltpu.SemaphoreType.DMA,
            pltpu.SemaphoreType.DMA,
        ],
    )
    return fn(ei3, el, er, *feats)


def kernel(features, edge_index, W1, b1, g1, be1, W2, b2, g2, be2,
           go_embed, go_bias, gat_W, attn_l, attn_r, gat_bias):
    row = lambda v: v.reshape(1, H)
    outs = _gat_fc_call(go_embed, gat_W, row(attn_l), row(attn_r))
    feats, el, er = outs[:NCHUNK], outs[NCHUNK], outs[NCHUNK + 1]
    sc_outs = _gat_aggregate(edge_index.reshape(2, ER, 128),
                             el.reshape(N // 16, 16), er.reshape(N // 16, 16),
                             feats)
    x = _mlp_call(features, W1, row(b1), row(g1), row(be1),
                  W2, row(b2), row(g2), row(be2))
    accs = sc_outs[:NCHUNK]
    return _logits_call(x, accs, sc_outs[NCHUNK], sc_outs[NCHUNK + 1],
                        row(gat_bias), go_bias)


# speedup vs baseline: 1.0014x; 1.0014x over previous
import jax
import jax.numpy as jnp
from jax import lax
from jax.experimental import pallas as pl
from jax.experimental.pallas import tpu as pltpu
from jax.experimental.pallas import tpu_sc as plsc

B = 1024
NB_IPRS = 26406
N = 10000
E = 160000
H = 512

KBLK = 512
KSTEPS = (NB_IPRS + KBLK - 1) // KBLK
KREM = NB_IPRS - (KSTEPS - 1) * KBLK

NBLK = 400
NSTEPS = N // NBLK

NCHUNK = 8
CW = H // NCHUNK

ER = E // 128
RPT = 80
RLAST = 50
CP = 624

N2 = 10240
NB2 = 512
NS2 = N2 // NB2


def _layer_norm(x, g, be):
    mu = jnp.mean(x, axis=-1, keepdims=True)
    var = jnp.mean(x * x, axis=-1, keepdims=True) - mu * mu
    return (x - mu) * lax.rsqrt(var + 1e-5) * g + be


def _mlp_kernel(f_ref, w1_ref, b1_ref, g1_ref, be1_ref, w2_ref, b2_ref,
                g2_ref, be2_ref, o_ref, acc_ref):
    k = pl.program_id(0)

    @pl.when(k == 0)
    def _():
        acc_ref[...] = jnp.zeros_like(acc_ref)

    lim = jnp.where(k == KSTEPS - 1, KREM, KBLK)
    col = lax.broadcasted_iota(jnp.int32, (B, KBLK), 1)
    row = lax.broadcasted_iota(jnp.int32, (KBLK, H), 0)
    fb = jnp.where(col < lim, f_ref[...], 0.0)
    wb = jnp.where(row < lim, w1_ref[...], 0.0)

    acc_ref[...] += jnp.dot(fb, wb, preferred_element_type=jnp.float32)

    @pl.when(k == KSTEPS - 1)
    def _():
        x1 = jax.nn.relu(acc_ref[...] + b1_ref[...])
        x1 = _layer_norm(x1, g1_ref[...], be1_ref[...])
        x2 = jax.nn.relu(
            jnp.dot(x1, w2_ref[...], preferred_element_type=jnp.float32)
            + b2_ref[...])
        x2 = _layer_norm(x2, g2_ref[...], be2_ref[...])
        o_ref[...] = x1 + x2


def _mlp_call(features, W1, b1, g1, be1, W2, b2, g2, be2):
    return pl.pallas_call(
        _mlp_kernel,
        grid=(KSTEPS,),
        in_specs=[
            pl.BlockSpec((B, KBLK), lambda k: (0, k)),
            pl.BlockSpec((KBLK, H), lambda k: (k, 0)),
        ] + [pl.BlockSpec((1, H), lambda k: (0, 0))] * 3
        + [pl.BlockSpec((H, H), lambda k: (0, 0))]
        + [pl.BlockSpec((1, H), lambda k: (0, 0))] * 3,
        out_specs=pl.BlockSpec((B, H), lambda k: (0, 0)),
        out_shape=jax.ShapeDtypeStruct((B, H), jnp.float32),
        scratch_shapes=[pltpu.VMEM((B, H), jnp.float32)],
    )(features, W1, b1, g1, be1, W2, b2, g2, be2)


def _gat_fc_kernel(ge_ref, w_ref, al_ref, ar_ref, *o_refs):
    m = pl.program_id(0)
    f_refs, el_ref, er_ref = o_refs[:NCHUNK], o_refs[NCHUNK], o_refs[NCHUNK + 1]
    fb = jnp.dot(ge_ref[...], w_ref[...], preferred_element_type=jnp.float32)
    for c in range(NCHUNK):
        f_refs[c][...] = fb[:, c * CW:(c + 1) * CW]
    el_ref[pl.ds(m * 25, 25), :] = jnp.sum(
        fb * al_ref[...], axis=-1).reshape(25, 16)
    er_ref[pl.ds(m * 25, 25), :] = jnp.sum(
        fb * ar_ref[...], axis=-1).reshape(25, 16)


def _gat_fc_call(go_embed, gat_W, attn_l, attn_r):
    chunk = jax.ShapeDtypeStruct((N, CW), jnp.float32)
    return pl.pallas_call(
        _gat_fc_kernel,
        grid=(NSTEPS,),
        in_specs=[
            pl.BlockSpec((NBLK, H), lambda m: (m, 0)),
            pl.BlockSpec((H, H), lambda m: (0, 0)),
            pl.BlockSpec((1, H), lambda m: (0, 0)),
            pl.BlockSpec((1, H), lambda m: (0, 0)),
        ],
        out_specs=[pl.BlockSpec((NBLK, CW), lambda m: (m, 0))] * NCHUNK
        + [pl.BlockSpec((N // 16, 16), lambda m: (0, 0))] * 2,
        out_shape=[chunk] * NCHUNK
        + [jax.ShapeDtypeStruct((N // 16, 16), jnp.float32)] * 2,
    )(go_embed, gat_W, attn_l, attn_r)


def _logits_kernel(x_ref, *refs):
    a_refs = refs[:NCHUNK]
    dn_ref, dn1_ref, gbias_ref, gob_ref, o_ref = refs[NCHUNK:]
    m = pl.program_id(0)
    gob = jnp.concatenate([a[...] for a in a_refs], axis=1)
    dn = dn_ref[...][:, 0:1] + dn1_ref[...][:, 0:1]
    gob = gob / (dn + 1e-9) + gbias_ref[...]
    z = lax.dot_general(x_ref[...], gob, (((1,), (1,)), ((), ())),
                        preferred_element_type=jnp.float32)
    o_ref[...] = jax.nn.sigmoid(z + gob_ref[pl.ds(m, 1), :])


def _logits_call(x, accs, denom16, denomb, gat_bias, go_bias):
    go_bias = jnp.pad(go_bias.reshape(N), (0, N2 - N)).reshape(NS2, NB2)
    out = pl.pallas_call(
        _logits_kernel,
        grid=(NS2,),
        in_specs=[pl.BlockSpec((B, H), lambda m: (0, 0))]
        + [pl.BlockSpec((NB2, CW), lambda m: (m, 0))] * NCHUNK
        + [
            pl.BlockSpec((NB2, CW), lambda m: (m, 0)),
            pl.BlockSpec((NB2, CW), lambda m: (m, 0)),
            pl.BlockSpec((1, H), lambda m: (0, 0)),
            pl.BlockSpec((NS2, NB2), lambda m: (0, 0)),
        ],
        out_specs=pl.BlockSpec((B, NB2), lambda m: (0, m)),
        out_shape=jax.ShapeDtypeStruct((B, N2), jnp.float32),
    )(x, *accs, denom16, denomb, gat_bias, go_bias)
    return out[:, :N]


def _splat(v):
    return jnp.full((16,), v, jnp.int32)


def _sc_body(ei_h, el_h, er_h, *refs):
    f_h = refs[:NCHUNK]
    a_h = refs[NCHUNK:2 * NCHUNK]
    den_h = refs[2 * NCHUNK]
    den1_h = refs[2 * NCHUNK + 1]
    (srcv, dstv, eev, elv, erv, buf, buf2, buf3, zbuf, acc_sh,
     sem, sem2, sem3, sem4, sem5, sem6) = refs[2 * NCHUNK + 2:]

    cid = lax.axis_index("c")
    sid = lax.axis_index("s")
    n = jnp.where(sid == 15, RLAST, RPT)
    p0 = sid * CP
    r0 = sid * RPT

    @pl.when(sid < 15)
    def _():
        pltpu.sync_copy(ei_h.at[0, pl.ds(r0, RPT), :], srcv)
        pltpu.sync_copy(ei_h.at[1, pl.ds(r0, RPT), :], dstv)

    @pl.when(sid == 15)
    def _():
        pltpu.sync_copy(ei_h.at[0, pl.ds(r0, RLAST), :],
                        srcv.at[pl.ds(0, RLAST)])
        pltpu.sync_copy(ei_h.at[1, pl.ds(r0, RLAST), :],
                        dstv.at[pl.ds(0, RLAST)])

    pltpu.sync_copy(el_h, elv)
    pltpu.sync_copy(er_h, erv)

    def ee_row(j, _):
        for i in range(8):
            sv = srcv[j, pl.ds(16 * i, 16)]
            dv = dstv[j, pl.ds(16 * i, 16)]
            e = (plsc.load_gather(elv, [sv >> 4, sv & 15])
                 + plsc.load_gather(erv, [dv >> 4, dv & 15]))
            e = jnp.maximum(e, 0.2 * e)
            eev[j, pl.ds(16 * i, 16)] = jnp.exp(e)
        return 0

    lax.fori_loop(0, n, ee_row, 0)

    def zb_row(i, _):
        for d in range(CW // 16):
            zbuf[i, pl.ds(16 * d, 16)] = jnp.zeros((16,), jnp.float32)
        return 0

    lax.fori_loop(0, 78, zb_row, 0)

    def zero_accsh():
        for t in range(8):
            pltpu.sync_copy(zbuf, acc_sh.at[pl.ds(p0 + 78 * t, 78)])

        @pl.when(sid == 15)
        def _():
            pltpu.sync_copy(zbuf.at[pl.ds(0, 16)],
                            acc_sh.at[pl.ds(N - 16, 16)])

    def scale_rows(b, j):
        def srow(g, _):
            r0 = g * 8
            for u in range(8):
                r = r0 + u
                eef = plsc.load_gather(eev, [_splat(j), _splat(r)])
                for d in range(CW // 16):
                    sl = pl.ds(16 * d, 16)
                    b[r, sl] = b[r, sl] * eef
            return 0

        lax.fori_loop(0, 16, srow, 0)

    def chunk_pass(feat_h):
        bufs = (buf, buf2, buf3)
        gsem = (sem, sem2, sem3)
        ssem = (sem4, sem5, sem6)

        def gissue(j, b):
            pltpu.async_copy(feat_h.at[srcv.at[j]], bufs[b], gsem[b])

        gissue(0, 0)
        gissue(1, 1)

        def grp(g, _):
            for u in range(3):
                j = 3 * g + u

                @pl.when(j < n)
                def _(j=j, u=u):
                    pltpu.make_async_copy(feat_h.at[srcv.at[j]], bufs[u],
                                          gsem[u]).wait()
                    scale_rows(bufs[u], j)
                    pltpu.async_copy(bufs[u], acc_sh.at[dstv.at[j]],
                                     ssem[u], add=True)

                @pl.when((j >= 1) & (j - 1 < n))
                def _(j=j, u=u):
                    b = (u + 2) % 3
                    pltpu.make_async_copy(bufs[b],
                                          acc_sh.at[dstv.at[j - 1]],
                                          ssem[b]).wait()

                @pl.when(j + 2 < n)
                def _(j=j, u=u):
                    gissue(j + 2, (u + 2) % 3)

            return 0

        lax.fori_loop(0, 27, grp, 0)

    def denom_pass(lo, hi):
        def blk(j, _):
            def row(g, _):
                for u in range(8):
                    r = g * 8 + u
                    eef = plsc.load_gather(eev, [_splat(j), _splat(r)])
                    for d in range(CW // 16):
                        buf[r, pl.ds(16 * d, 16)] = eef
                return 0

            lax.fori_loop(0, 16, row, 0)
            pltpu.sync_copy(buf, acc_sh.at[dstv.at[j]], add=True)
            return 0

        lax.fori_loop(lo, hi, blk, 0)

    def copy_out(src_sh, dst_ref):
        pltpu.sync_copy(src_sh.at[pl.ds(p0, CP)], dst_ref.at[pl.ds(p0, CP)])

        @pl.when(sid == 15)
        def _():
            pltpu.sync_copy(src_sh.at[pl.ds(N - 16, 16)],
                            dst_ref.at[pl.ds(N - 16, 16)])

    def run_sc(half, dref):
        for i in range(NCHUNK // 2):
            zero_accsh()
            plsc.subcore_barrier()
            chunk_pass(f_h[half * (NCHUNK // 2) + i])
            plsc.subcore_barrier()
            copy_out(acc_sh, a_h[half * (NCHUNK // 2) + i])
        zero_accsh()
        plsc.subcore_barrier()
        if half == 0:
            denom_pass(0, n // 2)
        else:
            denom_pass(n // 2, n)
        plsc.subcore_barrier()
        copy_out(acc_sh, dref)

    @pl.when(cid == 0)
    def _():
        run_sc(0, den_h)

    @pl.when(cid == 1)
    def _():
        run_sc(1, den1_h)


def _gat_aggregate(ei3, el, er, feats):
    mesh = plsc.VectorSubcoreMesh(core_axis_name="c", subcore_axis_name="s")
    chunk = jax.ShapeDtypeStruct((N2, CW), jnp.float32)
    out_type = [chunk] * NCHUNK + [jax.ShapeDtypeStruct((N2, CW), jnp.float32)] * 2
    fn = pl.kernel(
        _sc_body,
        out_type=out_type,
        mesh=mesh,
        compiler_params=pltpu.CompilerParams(needs_layout_passes=False,
                                             use_tc_tiling_on_sc=False),
        scratch_types=[
            pltpu.VMEM((RPT, 128), jnp.int32),
            pltpu.VMEM((RPT, 128), jnp.int32),
            pltpu.VMEM((RPT, 128), jnp.float32),
            pltpu.VMEM((N // 16, 16), jnp.float32),
            pltpu.VMEM((N // 16, 16), jnp.float32),
            pltpu.VMEM((128, CW), jnp.float32),
            pltpu.VMEM((128, CW), jnp.float32),
            pltpu.VMEM((128, CW), jnp.float32),
            pltpu.VMEM((78, CW), jnp.float32),
            pltpu.VMEM_SHARED((N, CW), jnp.float32),
            pltpu.SemaphoreType.DMA,
            pltpu.SemaphoreType.DMA,
            pltpu.SemaphoreType.DMA,
            pltpu.SemaphoreType.DMA,
            pltpu.SemaphoreType.DMA,
            pltpu.SemaphoreType.DMA,
        ],
    )
    return fn(ei3, el, er, *feats)


def kernel(features, edge_index, W1, b1, g1, be1, W2, b2, g2, be2,
           go_embed, go_bias, gat_W, attn_l, attn_r, gat_bias):
    row = lambda v: v.reshape(1, H)
    outs = _gat_fc_call(go_embed, gat_W, row(attn_l), row(attn_r))
    feats, el, er = outs[:NCHUNK], outs[NCHUNK], outs[NCHUNK + 1]
    sc_outs = _gat_aggregate(edge_index.reshape(2, ER, 128), el, er, feats)
    x = _mlp_call(features, W1, row(b1), row(g1), row(be1),
                  W2, row(b2), row(g2), row(be2))
    accs = sc_outs[:NCHUNK]
    return _logits_call(x, accs, sc_outs[NCHUNK], sc_outs[NCHUNK + 1],
                        row(gat_bias), go_bias)
